# baseline (speedup 1.0000x reference)
import functools

import jax
import jax.numpy as jnp
from jax import lax
from jax.experimental import pallas as pl
from jax.experimental.pallas import tpu as pltpu
from jax.experimental.pallas import tpu_sc as plsc

E = 800000
EMB_DIM = 64
REL_DIM = 16
N_REL = 1000
N_REL_PAD = 1024
GATHER_W = 128

NC = 2
NS = 16
NW = NC * NS

B_PER_W = E // NW
CHUNK = 128
FULL_CHUNKS = B_PER_W // CHUNK
TAIL = B_PER_W - FULL_CHUNKS * CHUNK
NBUF = 5
GROUPS = FULL_CHUNKS // NBUF


def _project_body(t_ref, w_ref, bias_ref, o_ref):
    wr = w_ref[EMB_DIM:EMB_DIM + REL_DIM, :]
    proj = jnp.dot(t_ref[...], wr, preferred_element_type=jnp.float32)
    proj = proj + bias_ref[...]
    proj = jnp.concatenate(
        [proj, jnp.zeros((N_REL_PAD - N_REL, EMB_DIM), jnp.float32)], axis=0)
    proj = jnp.concatenate(
        [proj, jnp.zeros((N_REL_PAD, GATHER_W - EMB_DIM), jnp.float32)], axis=1)
    o_ref[...] = proj


def _project_table(rel_table, W, bias2d):
    in_dim = 2 * EMB_DIM + REL_DIM
    return pl.pallas_call(
        _project_body,
        in_specs=[
            pl.BlockSpec((N_REL, REL_DIM), lambda: (0, 0)),
            pl.BlockSpec((in_dim, EMB_DIM), lambda: (0, 0)),
            pl.BlockSpec((1, EMB_DIM), lambda: (0, 0)),
        ],
        out_specs=pl.BlockSpec((N_REL_PAD, GATHER_W), lambda: (0, 0)),
        out_shape=jax.ShapeDtypeStruct((N_REL_PAD, GATHER_W), jnp.float32),
    )(rel_table, W, bias2d)


def _sc_gather(table_pad, rel_ids):
    mesh = plsc.VectorSubcoreMesh(core_axis_name="c", subcore_axis_name="s")

    @functools.partial(
        pl.kernel,
        mesh=mesh,
        out_type=jax.ShapeDtypeStruct((E, GATHER_W), jnp.float32),
        scratch_types=[
            pltpu.VMEM((B_PER_W,), jnp.int32),
            pltpu.VMEM((NBUF, CHUNK, GATHER_W), jnp.float32),
            pltpu.SemaphoreType.DMA((NBUF,)),
            pltpu.SemaphoreType.DMA((NBUF,)),
        ],
    )
    def k(table_hbm, idx_hbm, out_hbm, idx_all, bufs, gsem, ssem):
        wid = lax.axis_index("s") * NC + lax.axis_index("c")
        base = wid * B_PER_W
        pltpu.sync_copy(idx_hbm.at[pl.ds(base, B_PER_W)], idx_all)

        def start_gather(s, chunk):
            pltpu.async_copy(
                table_hbm.at[idx_all.at[pl.ds(chunk * CHUNK, CHUNK)]],
                bufs.at[s], gsem.at[s])

        def start_store(s, chunk):
            pltpu.async_copy(
                bufs.at[s], out_hbm.at[pl.ds(base + chunk * CHUNK, CHUNK)],
                ssem.at[s])

        def wait_gather(s):
            pltpu.make_async_copy(
                table_hbm.at[pl.ds(0, CHUNK)], bufs.at[s], gsem.at[s]).wait()

        def wait_store(s):
            pltpu.make_async_copy(
                bufs.at[s], out_hbm.at[pl.ds(base, CHUNK)], ssem.at[s]).wait()

        for s in range(NBUF):
            start_gather(s, s)

        @pl.loop(0, GROUPS - 1)
        def _(g):
            for s in range(NBUF):
                wait_gather(s)
                start_store(s, g * NBUF + s)
            for s in range(NBUF):
                wait_store(s)
                start_gather(s, (g + 1) * NBUF + s)

        for s in range(NBUF):
            wait_gather(s)
            start_store(s, (GROUPS - 1) * NBUF + s)

        wait_store(0)
        toff = FULL_CHUNKS * CHUNK
        tail_buf = bufs.at[0].at[pl.ds(0, TAIL)]
        tail_out = out_hbm.at[pl.ds(base + toff, TAIL)]
        pltpu.async_copy(
            table_hbm.at[idx_all.at[pl.ds(toff, TAIL)]], tail_buf, gsem.at[0])
        pltpu.make_async_copy(
            table_hbm.at[pl.ds(0, TAIL)], tail_buf, gsem.at[0]).wait()
        pltpu.async_copy(tail_buf, tail_out, ssem.at[0])
        pltpu.make_async_copy(tail_buf, tail_out, ssem.at[0]).wait()
        for s in range(1, NBUF):
            wait_store(s)

    return k(table_pad, rel_ids)


BE = 4000


def _tc_body(a_ref, rel_ref, b2_ref, w_ref, o_ref):
    wa = w_ref[0:EMB_DIM, :]
    wb = w_ref[EMB_DIM + REL_DIM:, :]
    acc = jnp.dot(a_ref[...], wa, preferred_element_type=jnp.float32)
    acc += jnp.dot(b2_ref[...], wb, preferred_element_type=jnp.float32)
    o_ref[...] = acc + rel_ref[:, 0:EMB_DIM]


def _tc_fused(emb_a, rel_g, emb_b, W):
    in_dim = 2 * EMB_DIM + REL_DIM
    grid = (E // BE,)
    return pl.pallas_call(
        _tc_body,
        grid=grid,
        in_specs=[
            pl.BlockSpec((BE, EMB_DIM), lambda i: (i, 0)),
            pl.BlockSpec((BE, GATHER_W), lambda i: (i, 0)),
            pl.BlockSpec((BE, EMB_DIM), lambda i: (i, 0)),
            pl.BlockSpec((in_dim, EMB_DIM), lambda i: (0, 0)),
        ],
        out_specs=pl.BlockSpec((BE, EMB_DIM), lambda i: (i, 0)),
        out_shape=jax.ShapeDtypeStruct((E, EMB_DIM), jnp.float32),
        compiler_params=pltpu.CompilerParams(
            dimension_semantics=("arbitrary",),
        ),
    )(emb_a, rel_g, emb_b, W)


def kernel(emb_a, rel_ids, emb_b, rel_table, W, b):
    table_pad = _project_table(rel_table, W, b.reshape(1, EMB_DIM))
    rel_g = _sc_gather(table_pad, rel_ids)
    return _tc_fused(emb_a, rel_g, emb_b, W)


# speedup vs baseline: 1.0974x; 1.0974x over previous
import functools

import jax
import jax.numpy as jnp
from jax import lax
from jax.experimental import pallas as pl
from jax.experimental.pallas import tpu as pltpu
from jax.experimental.pallas import tpu_sc as plsc

E = 800000
EMB_DIM = 64
REL_DIM = 16
N_REL = 1000
N_REL_PAD = 1024

NC = 2
NS = 16
NW = NC * NS
L = 16

B_PER_W = E // NW
CHUNK = 256
FULL_CHUNKS = B_PER_W // CHUNK
TAIL_E = B_PER_W - FULL_CHUNKS * CHUNK


def _project_body(t_ref, w_ref, bias_ref, o_ref):
    wr = w_ref[EMB_DIM:EMB_DIM + REL_DIM, :]
    proj = jnp.dot(t_ref[...], wr, preferred_element_type=jnp.float32)
    proj = proj + bias_ref[...]
    o_ref[...] = jnp.concatenate(
        [proj, jnp.zeros((N_REL_PAD - N_REL, EMB_DIM), jnp.float32)], axis=0)


def _project_table(rel_table, W, bias2d):
    in_dim = 2 * EMB_DIM + REL_DIM
    return pl.pallas_call(
        _project_body,
        in_specs=[
            pl.BlockSpec((N_REL, REL_DIM), lambda: (0, 0)),
            pl.BlockSpec((in_dim, EMB_DIM), lambda: (0, 0)),
            pl.BlockSpec((1, EMB_DIM), lambda: (0, 0)),
        ],
        out_specs=pl.BlockSpec((N_REL_PAD, EMB_DIM), lambda: (0, 0)),
        out_shape=jax.ShapeDtypeStruct((N_REL_PAD, EMB_DIM), jnp.float32),
    )(rel_table, W, bias2d)


def _sc_gather(table_pad, rel_ids):
    mesh = plsc.VectorSubcoreMesh(core_axis_name="c", subcore_axis_name="s")

    @functools.partial(
        pl.kernel,
        mesh=mesh,
        out_type=jax.ShapeDtypeStruct((E, EMB_DIM), jnp.float32),
        compiler_params=pltpu.CompilerParams(needs_layout_passes=False),
        scratch_types=[
            pltpu.VMEM((N_REL_PAD * EMB_DIM,), jnp.float32),
            pltpu.VMEM((B_PER_W,), jnp.int32),
            pltpu.VMEM((CHUNK, EMB_DIM), jnp.float32),
        ],
    )
    def k(table_hbm, idx_hbm, out_hbm, table_v, idx_v, stg):
        wid = lax.axis_index("s") * NC + lax.axis_index("c")
        base = wid * B_PER_W
        pltpu.sync_copy(table_hbm, table_v)
        pltpu.sync_copy(idx_hbm.at[pl.ds(base, B_PER_W)], idx_v)
        lanes = lax.iota(jnp.int32, L)
        zero16 = lanes * 0

        def gather_edge(goff, row):
            ids16 = plsc.load_gather(idx_v, [zero16 + goff])
            tbase = ids16 * EMB_DIM
            for kq in range(EMB_DIM // L):
                v = plsc.load_gather(table_v, [tbase + (lanes + kq * L)])
                stg[row, pl.ds(kq * L, L)] = v

        @pl.loop(0, FULL_CHUNKS)
        def _(c):
            @pl.loop(0, CHUNK)
            def _(e):
                gather_edge(c * CHUNK + e, e)
            pltpu.sync_copy(stg, out_hbm.at[pl.ds(base + c * CHUNK, CHUNK)])

        toff = FULL_CHUNKS * CHUNK

        @pl.loop(0, TAIL_E)
        def _(e):
            gather_edge(toff + e, e)

        pltpu.sync_copy(stg.at[pl.ds(0, TAIL_E)],
                        out_hbm.at[pl.ds(base + toff, TAIL_E)])

    return k(table_pad, rel_ids)


BE = 4000


def _tc_body(a_ref, rel_ref, b2_ref, w_ref, o_ref):
    wa = w_ref[0:EMB_DIM, :]
    wb = w_ref[EMB_DIM + REL_DIM:, :]
    acc = jnp.dot(a_ref[...], wa, preferred_element_type=jnp.float32)
    acc += jnp.dot(b2_ref[...], wb, preferred_element_type=jnp.float32)
    o_ref[...] = acc + rel_ref[...]


def _tc_fused(emb_a, rel_g, emb_b, W):
    in_dim = 2 * EMB_DIM + REL_DIM
    grid = (E // BE,)
    return pl.pallas_call(
        _tc_body,
        grid=grid,
        in_specs=[
            pl.BlockSpec((BE, EMB_DIM), lambda i: (i, 0)),
            pl.BlockSpec((BE, EMB_DIM), lambda i: (i, 0)),
            pl.BlockSpec((BE, EMB_DIM), lambda i: (i, 0)),
            pl.BlockSpec((in_dim, EMB_DIM), lambda i: (0, 0)),
        ],
        out_specs=pl.BlockSpec((BE, EMB_DIM), lambda i: (i, 0)),
        out_shape=jax.ShapeDtypeStruct((E, EMB_DIM), jnp.float32),
        compiler_params=pltpu.CompilerParams(
            dimension_semantics=("arbitrary",),
        ),
    )(emb_a, rel_g, emb_b, W)


def kernel(emb_a, rel_ids, emb_b, rel_table, W, b):
    table_pad = _project_table(rel_table, W, b.reshape(1, EMB_DIM))
    rel_g = _sc_gather(table_pad.reshape(N_REL_PAD * EMB_DIM), rel_ids)
    return _tc_fused(emb_a, rel_g, emb_b, W)


# speedup vs baseline: 1.1534x; 1.0510x over previous
import functools

import jax
import jax.numpy as jnp
from jax import lax
from jax.experimental import pallas as pl
from jax.experimental.pallas import tpu as pltpu
from jax.experimental.pallas import tpu_sc as plsc

E = 800000
EMB_DIM = 64
REL_DIM = 16
N_REL = 1000
N_REL_PAD = 1024

NC = 2
NS = 16
NW = NC * NS
L = 16

B_PER_W = E // NW
CHUNK = 256
FULL_CHUNKS = B_PER_W // CHUNK
TAIL_E = B_PER_W - FULL_CHUNKS * CHUNK


def _project_body(t_ref, w_ref, bias_ref, o_ref):
    wr = w_ref[EMB_DIM:EMB_DIM + REL_DIM, :]
    proj = jnp.dot(t_ref[...], wr, preferred_element_type=jnp.float32)
    proj = proj + bias_ref[...]
    o_ref[...] = jnp.concatenate(
        [proj, jnp.zeros((N_REL_PAD - N_REL, EMB_DIM), jnp.float32)], axis=0)


def _project_table(rel_table, W, bias2d):
    in_dim = 2 * EMB_DIM + REL_DIM
    return pl.pallas_call(
        _project_body,
        in_specs=[
            pl.BlockSpec((N_REL, REL_DIM), lambda: (0, 0)),
            pl.BlockSpec((in_dim, EMB_DIM), lambda: (0, 0)),
            pl.BlockSpec((1, EMB_DIM), lambda: (0, 0)),
        ],
        out_specs=pl.BlockSpec((N_REL_PAD, EMB_DIM), lambda: (0, 0)),
        out_shape=jax.ShapeDtypeStruct((N_REL_PAD, EMB_DIM), jnp.float32),
    )(rel_table, W, bias2d)


def _sc_gather(table_pad, rel_ids):
    mesh = plsc.VectorSubcoreMesh(core_axis_name="c", subcore_axis_name="s")

    @functools.partial(
        pl.kernel,
        mesh=mesh,
        out_type=jax.ShapeDtypeStruct((E, EMB_DIM), jnp.float32),
        compiler_params=pltpu.CompilerParams(needs_layout_passes=False),
        scratch_types=[
            pltpu.VMEM((N_REL_PAD * EMB_DIM,), jnp.float32),
            pltpu.VMEM((B_PER_W,), jnp.int32),
            pltpu.VMEM((CHUNK, EMB_DIM), jnp.float32),
        ],
    )
    def k(table_hbm, idx_hbm, out_hbm, table_v, idx_v, stg):
        wid = lax.axis_index("s") * NC + lax.axis_index("c")
        base = wid * B_PER_W
        pltpu.sync_copy(table_hbm, table_v)
        pltpu.sync_copy(idx_hbm.at[pl.ds(base, B_PER_W)], idx_v)
        lanes = lax.iota(jnp.int32, L)
        zero16 = lanes * 0

        def gather_edge(goff, row):
            ids16 = plsc.load_gather(idx_v, [zero16 + goff])
            tbase = ids16 * EMB_DIM
            for kq in range(EMB_DIM // L):
                v = plsc.load_gather(table_v, [tbase + (lanes + kq * L)])
                stg[row, pl.ds(kq * L, L)] = v

        @pl.loop(0, FULL_CHUNKS)
        def _(c):
            @plsc.parallel_loop(0, CHUNK, unroll=8)
            def _(e):
                gather_edge(c * CHUNK + e, e)
            pltpu.sync_copy(stg, out_hbm.at[pl.ds(base + c * CHUNK, CHUNK)])

        toff = FULL_CHUNKS * CHUNK

        @plsc.parallel_loop(0, TAIL_E, unroll=8)
        def _(e):
            gather_edge(toff + e, e)

        pltpu.sync_copy(stg.at[pl.ds(0, TAIL_E)],
                        out_hbm.at[pl.ds(base + toff, TAIL_E)])

    return k(table_pad, rel_ids)


BE = 4000


def _tc_body(a_ref, rel_ref, b2_ref, w_ref, o_ref):
    wa = w_ref[0:EMB_DIM, :]
    wb = w_ref[EMB_DIM + REL_DIM:, :]
    acc = jnp.dot(a_ref[...], wa, preferred_element_type=jnp.float32)
    acc += jnp.dot(b2_ref[...], wb, preferred_element_type=jnp.float32)
    o_ref[...] = acc + rel_ref[...]


def _tc_fused(emb_a, rel_g, emb_b, W):
    in_dim = 2 * EMB_DIM + REL_DIM
    grid = (E // BE,)
    return pl.pallas_call(
        _tc_body,
        grid=grid,
        in_specs=[
            pl.BlockSpec((BE, EMB_DIM), lambda i: (i, 0)),
            pl.BlockSpec((BE, EMB_DIM), lambda i: (i, 0)),
            pl.BlockSpec((BE, EMB_DIM), lambda i: (i, 0)),
            pl.BlockSpec((in_dim, EMB_DIM), lambda i: (0, 0)),
        ],
        out_specs=pl.BlockSpec((BE, EMB_DIM), lambda i: (i, 0)),
        out_shape=jax.ShapeDtypeStruct((E, EMB_DIM), jnp.float32),
        compiler_params=pltpu.CompilerParams(
            dimension_semantics=("arbitrary",),
        ),
    )(emb_a, rel_g, emb_b, W)


def kernel(emb_a, rel_ids, emb_b, rel_table, W, b):
    table_pad = _project_table(rel_table, W, b.reshape(1, EMB_DIM))
    rel_g = _sc_gather(table_pad.reshape(N_REL_PAD * EMB_DIM), rel_ids)
    return _tc_fused(emb_a, rel_g, emb_b, W)


# speedup vs baseline: 1.1593x; 1.0052x over previous
import functools

import jax
import jax.numpy as jnp
from jax import lax
from jax.experimental import pallas as pl
from jax.experimental.pallas import tpu as pltpu
from jax.experimental.pallas import tpu_sc as plsc

E = 800000
EMB_DIM = 64
REL_DIM = 16
N_REL = 1000
N_REL_PAD = 1024

NC = 2
NS = 16
NW = NC * NS
L = 16

B_PER_W = E // NW
CHUNK = 256
FULL_CHUNKS = B_PER_W // CHUNK
TAIL_E = B_PER_W - FULL_CHUNKS * CHUNK


def _project_body(t_ref, w_ref, bias_ref, o_ref):
    wr = w_ref[EMB_DIM:EMB_DIM + REL_DIM, :]
    proj = jnp.dot(t_ref[...], wr, preferred_element_type=jnp.float32)
    proj = proj + bias_ref[...]
    o_ref[...] = jnp.concatenate(
        [proj, jnp.zeros((N_REL_PAD - N_REL, EMB_DIM), jnp.float32)], axis=0)


def _project_table(rel_table, W, bias2d):
    in_dim = 2 * EMB_DIM + REL_DIM
    return pl.pallas_call(
        _project_body,
        in_specs=[
            pl.BlockSpec((N_REL, REL_DIM), lambda: (0, 0)),
            pl.BlockSpec((in_dim, EMB_DIM), lambda: (0, 0)),
            pl.BlockSpec((1, EMB_DIM), lambda: (0, 0)),
        ],
        out_specs=pl.BlockSpec((N_REL_PAD, EMB_DIM), lambda: (0, 0)),
        out_shape=jax.ShapeDtypeStruct((N_REL_PAD, EMB_DIM), jnp.float32),
    )(rel_table, W, bias2d)


def _sc_gather(table_pad, rel_ids):
    mesh = plsc.VectorSubcoreMesh(core_axis_name="c", subcore_axis_name="s")

    @functools.partial(
        pl.kernel,
        mesh=mesh,
        out_type=jax.ShapeDtypeStruct((E, EMB_DIM), jnp.float32),
        compiler_params=pltpu.CompilerParams(needs_layout_passes=False),
        scratch_types=[
            pltpu.VMEM((N_REL_PAD * EMB_DIM,), jnp.float32),
            pltpu.VMEM((B_PER_W,), jnp.int32),
            pltpu.VMEM((CHUNK, EMB_DIM), jnp.float32),
        ],
    )
    def k(table_hbm, idx_hbm, out_hbm, table_v, idx_v, stg):
        wid = lax.axis_index("s") * NC + lax.axis_index("c")
        base = wid * B_PER_W
        pltpu.sync_copy(table_hbm, table_v)
        pltpu.sync_copy(idx_hbm.at[pl.ds(base, B_PER_W)], idx_v)
        lanes = lax.iota(jnp.int32, L)
        zero16 = lanes * 0

        def gather_edge(goff, row):
            ids16 = plsc.load_gather(idx_v, [zero16 + goff])
            tbase = ids16 * EMB_DIM
            for kq in range(EMB_DIM // L):
                v = plsc.load_gather(table_v, [tbase + (lanes + kq * L)])
                stg[row, pl.ds(kq * L, L)] = v

        @pl.loop(0, FULL_CHUNKS)
        def _(c):
            @plsc.parallel_loop(0, CHUNK, unroll=8)
            def _(e):
                gather_edge(c * CHUNK + e, e)
            pltpu.sync_copy(stg, out_hbm.at[pl.ds(base + c * CHUNK, CHUNK)])

        toff = FULL_CHUNKS * CHUNK

        @plsc.parallel_loop(0, TAIL_E, unroll=8)
        def _(e):
            gather_edge(toff + e, e)

        pltpu.sync_copy(stg.at[pl.ds(0, TAIL_E)],
                        out_hbm.at[pl.ds(base + toff, TAIL_E)])

    return k(table_pad, rel_ids)


BE = 8000


def _tc_body(a_ref, rel_ref, b2_ref, w_ref, o_ref):
    wa = w_ref[0:EMB_DIM, :]
    wb = w_ref[EMB_DIM + REL_DIM:, :]
    acc = jnp.dot(a_ref[...], wa, preferred_element_type=jnp.float32)
    acc += jnp.dot(b2_ref[...], wb, preferred_element_type=jnp.float32)
    o_ref[...] = acc + rel_ref[...]


def _tc_fused(emb_a, rel_g, emb_b, W):
    in_dim = 2 * EMB_DIM + REL_DIM
    grid = (E // BE,)
    return pl.pallas_call(
        _tc_body,
        grid=grid,
        in_specs=[
            pl.BlockSpec((BE, EMB_DIM), lambda i: (i, 0)),
            pl.BlockSpec((BE, EMB_DIM), lambda i: (i, 0)),
            pl.BlockSpec((BE, EMB_DIM), lambda i: (i, 0)),
            pl.BlockSpec((in_dim, EMB_DIM), lambda i: (0, 0)),
        ],
        out_specs=pl.BlockSpec((BE, EMB_DIM), lambda i: (i, 0)),
        out_shape=jax.ShapeDtypeStruct((E, EMB_DIM), jnp.float32),
        compiler_params=pltpu.CompilerParams(
            dimension_semantics=("parallel",),
        ),
    )(emb_a, rel_g, emb_b, W)


def kernel(emb_a, rel_ids, emb_b, rel_table, W, b):
    table_pad = _project_table(rel_table, W, b.reshape(1, EMB_DIM))
    rel_g = _sc_gather(table_pad.reshape(N_REL_PAD * EMB_DIM), rel_ids)
    return _tc_fused(emb_a, rel_g, emb_b, W)
